# baseline (speedup 1.0000x reference)
import jax
import jax.numpy as jnp
from jax import lax
from jax.experimental import pallas as pl
from jax.experimental.pallas import tpu as pltpu
from jax.experimental.pallas import tpu_sc as plsc

B, S, D = 16, 2048, 128
T = 1024
NC, NS = 2, 16
L = 16
TR = 256
NASSIGN = (B * T) // TR
CH = 128
NCH = S // CH


def _body(hid, idx, out, idxv, datab, acc):
    c = lax.axis_index("c")
    w = lax.axis_index("s")
    tid = c * NS + w

    for a_rel in range(NASSIGN // (NC * NS)):
        a = tid * (NASSIGN // (NC * NS)) + a_rel
        b = a // (T // TR)
        t0 = (a % (T // TR)) * TR

        pltpu.sync_copy(idx.at[pl.ds(b * S, S)], idxv.at[pl.ds(0, S)])

        def _search(target):
            def bs(_, lh):
                lo, hi = lh
                mid = (lo + hi) // 2
                v = idxv[pl.ds(mid, L)][0]
                go = v < target
                return (jnp.where(go, mid + 1, lo), jnp.where(go, hi, mid))
            lo, _ = lax.fori_loop(0, 11, bs, (jnp.int32(0), jnp.int32(S)))
            return lo

        lo_s = _search(t0)
        hi_s = _search(t0 + TR)
        k0 = lo_s // CH
        kmax = (hi_s + CH - 1) // CH

        def _za(r, _):
            acc[pl.ds(r * L, L)] = jnp.zeros((L,), jnp.float32)
            return 0
        lax.fori_loop(0, (TR * D) // L, _za, 0)

        z = jnp.zeros((L,), jnp.float32)

        def _chunk(kk, carry):
            off = b * S + (k0 + kk) * CH
            pltpu.sync_copy(hid.at[pl.ds(off, CH)], datab)

            def _row(i, cr):
                prev, ptarg, cf, a0, a1, a2, a3, a4, a5, a6, a7 = cr
                t = idxv[pl.ds((k0 + kk) * CH + i, L)][0]
                new = t != prev
                rel = t - t0
                targ = jnp.where(jnp.logical_and(rel >= 0, rel < TR), rel, TR)

                @pl.when(new)
                def _flush():
                    rcp = jnp.ones((L,), jnp.float32) / jnp.full((L,), cf, jnp.float32)
                    for j, aj in enumerate((a0, a1, a2, a3, a4, a5, a6, a7)):
                        acc[pl.ds(ptarg * D + j * L, L)] = aj * rcp

                sf = jnp.where(new, 0.0, 1.0)
                avs = []
                for j, aj in enumerate((a0, a1, a2, a3, a4, a5, a6, a7)):
                    avs.append(datab[i, pl.ds(j * L, L)] + aj * sf)
                return (t, targ, 1.0 + cf * sf, *avs)

            return lax.fori_loop(0, CH, _row, carry)

        fin = lax.fori_loop(0, kmax - k0, _chunk,
                            (jnp.int32(-1), jnp.int32(TR), jnp.float32(1.0),
                             z, z, z, z, z, z, z, z))
        frcp = jnp.ones((L,), jnp.float32) / jnp.full((L,), fin[2], jnp.float32)
        for j in range(D // L):
            acc[pl.ds(fin[1] * D + j * L, L)] = fin[3 + j] * frcp

        pltpu.sync_copy(acc.at[pl.ds(0, TR * D)], out.at[pl.ds((b * T + t0) * D, TR * D)])


@jax.jit
def _aggregate(hidden, idx32):
    mesh = plsc.VectorSubcoreMesh(
        core_axis_name="c", subcore_axis_name="s", num_cores=NC, num_subcores=NS
    )
    out = pl.kernel(
        _body,
        out_type=jax.ShapeDtypeStruct((B * T * D,), jnp.float32),
        mesh=mesh,
        scratch_types=[
            pltpu.VMEM((S + L,), jnp.int32),
            pltpu.VMEM((CH, D), jnp.float32),
            pltpu.VMEM(((TR + 1) * D,), jnp.float32),
        ],
    )(hidden.reshape(B * S, D), idx32.reshape(B * S))
    return out.reshape(B, T, D)


def kernel(hidden, ori_indexes):
    return _aggregate(hidden, ori_indexes.astype(jnp.int32))


# speedup vs baseline: 1.2264x; 1.2264x over previous
import jax
import jax.numpy as jnp
from jax import lax
from jax.experimental import pallas as pl
from jax.experimental.pallas import tpu as pltpu
from jax.experimental.pallas import tpu_sc as plsc

B, S, D = 16, 2048, 128
T = 1024
NC, NS = 2, 16
L = 16
TR = 256
NASSIGN = (B * T) // TR
CH = 128
NCH = S // CH


def _body(hid, idx, out, idxv, datab, acc):
    c = lax.axis_index("c")
    w = lax.axis_index("s")
    tid = c * NS + w

    for a_rel in range(NASSIGN // (NC * NS)):
        a = tid * (NASSIGN // (NC * NS)) + a_rel
        b = a // (T // TR)
        t0 = (a % (T // TR)) * TR

        pltpu.sync_copy(idx.at[pl.ds(b * S, S)], idxv.at[pl.ds(0, S)])

        def _search(target):
            def bs(_, lh):
                lo, hi = lh
                mid = (lo + hi) // 2
                v = idxv[pl.ds(mid, L)][0]
                go = v < target
                return (jnp.where(go, mid + 1, lo), jnp.where(go, hi, mid))
            lo, _ = lax.fori_loop(0, 11, bs, (jnp.int32(0), jnp.int32(S)))
            return lo

        lo_s = _search(t0)
        hi_s = _search(t0 + TR)
        k0 = lo_s // CH
        kmax = (hi_s + CH - 1) // CH

        def _za(r, _):
            for j in range(D // L):
                acc[pl.ds(r * D + j * L, L)] = jnp.zeros((L,), jnp.float32)
            return 0
        lax.fori_loop(0, TR, _za, 0)

        z = jnp.zeros((L,), jnp.float32)

        def _chunk(kk, carry):
            off = b * S + (k0 + kk) * CH
            pltpu.sync_copy(hid.at[pl.ds(off, CH)], datab)

            def _row(i, cr):
                prev, ptarg, cf, a0, a1, a2, a3, a4, a5, a6, a7 = cr
                t = idxv[pl.ds((k0 + kk) * CH + i, L)][0]
                new = t != prev
                rel = t - t0
                targ = jnp.where(jnp.logical_and(rel >= 0, rel < TR), rel, TR)

                @pl.when(new)
                def _flush():
                    rcp = jnp.ones((L,), jnp.float32) / jnp.full((L,), cf, jnp.float32)
                    for j, aj in enumerate((a0, a1, a2, a3, a4, a5, a6, a7)):
                        acc[pl.ds(ptarg * D + j * L, L)] = aj * rcp

                sf = jnp.where(new, 0.0, 1.0)
                avs = []
                for j, aj in enumerate((a0, a1, a2, a3, a4, a5, a6, a7)):
                    avs.append(datab[i, pl.ds(j * L, L)] + aj * sf)
                return (t, targ, 1.0 + cf * sf, *avs)

            return lax.fori_loop(0, CH, _row, carry)

        fin = lax.fori_loop(0, kmax - k0, _chunk,
                            (jnp.int32(-1), jnp.int32(TR), jnp.float32(1.0),
                             z, z, z, z, z, z, z, z))
        frcp = jnp.ones((L,), jnp.float32) / jnp.full((L,), fin[2], jnp.float32)
        for j in range(D // L):
            acc[pl.ds(fin[1] * D + j * L, L)] = fin[3 + j] * frcp

        pltpu.sync_copy(acc.at[pl.ds(0, TR * D)], out.at[pl.ds((b * T + t0) * D, TR * D)])


@jax.jit
def _aggregate(hidden, idx32):
    mesh = plsc.VectorSubcoreMesh(
        core_axis_name="c", subcore_axis_name="s", num_cores=NC, num_subcores=NS
    )
    out = pl.kernel(
        _body,
        out_type=jax.ShapeDtypeStruct((B * T * D,), jnp.float32),
        mesh=mesh,
        scratch_types=[
            pltpu.VMEM((S + L,), jnp.int32),
            pltpu.VMEM((CH, D), jnp.float32),
            pltpu.VMEM(((TR + 1) * D,), jnp.float32),
        ],
    )(hidden.reshape(B * S, D), idx32.reshape(B * S))
    return out.reshape(B, T, D)


def kernel(hidden, ori_indexes):
    return _aggregate(hidden, ori_indexes.astype(jnp.int32))


# speedup vs baseline: 1.5543x; 1.2674x over previous
import jax
import jax.numpy as jnp
from jax import lax
from jax.experimental import pallas as pl
from jax.experimental.pallas import tpu as pltpu
from jax.experimental.pallas import tpu_sc as plsc

B, S, D = 16, 2048, 128
T = 1024
NC, NS = 2, 16
L = 16
TR = 256
NASSIGN = (B * T) // TR
CH = 128
NCH = S // CH


def _body(hid, idx, out, idxv, datab, acc):
    c = lax.axis_index("c")
    w = lax.axis_index("s")
    tid = c * NS + w

    for a_rel in range(NASSIGN // (NC * NS)):
        a = tid * (NASSIGN // (NC * NS)) + a_rel
        b = a // (T // TR)
        t0 = (a % (T // TR)) * TR

        pltpu.sync_copy(idx.at[pl.ds(b * S, S)], idxv.at[pl.ds(0, S)])

        def _search(target):
            def bs(_, lh):
                lo, hi = lh
                mid = (lo + hi) // 2
                v = idxv[pl.ds(mid, L)][0]
                go = v < target
                return (jnp.where(go, mid + 1, lo), jnp.where(go, hi, mid))
            lo, _ = lax.fori_loop(0, 11, bs, (jnp.int32(0), jnp.int32(S)))
            return lo

        lo_s = _search(t0)
        hi_s = _search(t0 + TR)
        k0 = lo_s // CH
        kmax = (hi_s + CH - 1) // CH

        def _za(r, _):
            for j in range(D // L):
                acc[pl.ds(r * D + j * L, L)] = jnp.zeros((L,), jnp.float32)
            return 0
        lax.fori_loop(0, TR, _za, 0)

        z = jnp.zeros((L,), jnp.float32)

        def _chunk(kk, carry):
            off = b * S + (k0 + kk) * CH
            pltpu.sync_copy(hid.at[pl.ds(off, CH)], datab)

            def _row(ib, cr):
                iv = idxv[pl.ds((k0 + kk) * CH + ib * 8, L)]
                for k in range(8):
                    prev, ptarg, cf, a0, a1, a2, a3, a4, a5, a6, a7 = cr
                    t = iv[k]
                    new = t != prev
                    rel = t - t0
                    targ = jnp.where(jnp.logical_and(rel >= 0, rel < TR), rel, TR)

                    @pl.when(new)
                    def _flush(vals=(a0, a1, a2, a3, a4, a5, a6, a7),
                               pt=ptarg, pc=cf):
                        rcp = jnp.ones((L,), jnp.float32) / jnp.full((L,), pc, jnp.float32)
                        for j, aj in enumerate(vals):
                            acc[pl.ds(pt * D + j * L, L)] = aj * rcp

                    sf = jnp.where(new, 0.0, 1.0)
                    i = ib * 8 + k
                    avs = []
                    for j, aj in enumerate((a0, a1, a2, a3, a4, a5, a6, a7)):
                        avs.append(datab[i, pl.ds(j * L, L)] + aj * sf)
                    cr = (t, targ, 1.0 + cf * sf, *avs)
                return cr

            return lax.fori_loop(0, CH // 8, _row, carry)

        fin = lax.fori_loop(0, kmax - k0, _chunk,
                            (jnp.int32(-1), jnp.int32(TR), jnp.float32(1.0),
                             z, z, z, z, z, z, z, z))
        frcp = jnp.ones((L,), jnp.float32) / jnp.full((L,), fin[2], jnp.float32)
        for j in range(D // L):
            acc[pl.ds(fin[1] * D + j * L, L)] = fin[3 + j] * frcp

        pltpu.sync_copy(acc.at[pl.ds(0, TR * D)], out.at[pl.ds((b * T + t0) * D, TR * D)])


@jax.jit
def _aggregate(hidden, idx32):
    mesh = plsc.VectorSubcoreMesh(
        core_axis_name="c", subcore_axis_name="s", num_cores=NC, num_subcores=NS
    )
    out = pl.kernel(
        _body,
        out_type=jax.ShapeDtypeStruct((B * T * D,), jnp.float32),
        mesh=mesh,
        scratch_types=[
            pltpu.VMEM((S + L,), jnp.int32),
            pltpu.VMEM((CH, D), jnp.float32),
            pltpu.VMEM(((TR + 1) * D,), jnp.float32),
        ],
    )(hidden.reshape(B * S, D), idx32.reshape(B * S))
    return out.reshape(B, T, D)


def kernel(hidden, ori_indexes):
    return _aggregate(hidden, ori_indexes.astype(jnp.int32))


# speedup vs baseline: 1.5865x; 1.0207x over previous
import jax
import jax.numpy as jnp
from jax import lax
from jax.experimental import pallas as pl
from jax.experimental.pallas import tpu as pltpu
from jax.experimental.pallas import tpu_sc as plsc

B, S, D = 16, 2048, 128
T = 1024
NC, NS = 2, 16
L = 16
TR = 256
NASSIGN = (B * T) // TR
CH = 128
NCH = S // CH


def _body(hid, idx, out, idxv, datab, acc):
    c = lax.axis_index("c")
    w = lax.axis_index("s")
    tid = c * NS + w

    for a_rel in range(NASSIGN // (NC * NS)):
        a = tid * (NASSIGN // (NC * NS)) + a_rel
        b = a // (T // TR)
        t0 = (a % (T // TR)) * TR

        pltpu.sync_copy(idx.at[pl.ds(b * S, S)], idxv.at[pl.ds(0, S)])

        def _search(target):
            def bs(_, lh):
                lo, hi = lh
                mid = (lo + hi) // 2
                v = idxv[pl.ds(mid, L)][0]
                go = v < target
                return (jnp.where(go, mid + 1, lo), jnp.where(go, hi, mid))
            lo, _ = lax.fori_loop(0, 11, bs, (jnp.int32(0), jnp.int32(S)))
            return lo

        lo_s = _search(t0)
        hi_s = _search(t0 + TR)
        k0 = lo_s // CH
        kmax = (hi_s + CH - 1) // CH

        def _za(r, _):
            for j in range(D // L):
                acc[pl.ds(r * D + j * L, L)] = jnp.zeros((L,), jnp.float32)
            return 0
        lax.fori_loop(0, TR, _za, 0)

        z = jnp.zeros((L,), jnp.float32)

        def _chunk(kk, carry):
            off = b * S + (k0 + kk) * CH
            pltpu.sync_copy(hid.at[pl.ds(off, CH)], datab)

            def _row(ib, cr):
                iv = idxv[pl.ds((k0 + kk) * CH + ib * L, L)]
                for k in range(L):
                    prev, ptarg, cf, a0, a1, a2, a3, a4, a5, a6, a7 = cr
                    t = iv[k]
                    new = t != prev
                    rel = t - t0
                    targ = jnp.where(jnp.logical_and(rel >= 0, rel < TR), rel, TR)

                    @pl.when(new)
                    def _flush(vals=(a0, a1, a2, a3, a4, a5, a6, a7),
                               pt=ptarg, pc=cf):
                        rcp = jnp.ones((L,), jnp.float32) / jnp.full((L,), pc, jnp.float32)
                        for j, aj in enumerate(vals):
                            acc[pl.ds(pt * D + j * L, L)] = aj * rcp

                    sf = jnp.where(new, 0.0, 1.0)
                    i = ib * L + k
                    avs = []
                    for j, aj in enumerate((a0, a1, a2, a3, a4, a5, a6, a7)):
                        avs.append(datab[i, pl.ds(j * L, L)] + aj * sf)
                    cr = (t, targ, 1.0 + cf * sf, *avs)
                return cr

            return lax.fori_loop(0, CH // L, _row, carry)

        fin = lax.fori_loop(0, kmax - k0, _chunk,
                            (jnp.int32(-1), jnp.int32(TR), jnp.float32(1.0),
                             z, z, z, z, z, z, z, z))
        frcp = jnp.ones((L,), jnp.float32) / jnp.full((L,), fin[2], jnp.float32)
        for j in range(D // L):
            acc[pl.ds(fin[1] * D + j * L, L)] = fin[3 + j] * frcp

        pltpu.sync_copy(acc.at[pl.ds(0, TR * D)], out.at[pl.ds((b * T + t0) * D, TR * D)])


@jax.jit
def _aggregate(hidden, idx32):
    mesh = plsc.VectorSubcoreMesh(
        core_axis_name="c", subcore_axis_name="s", num_cores=NC, num_subcores=NS
    )
    out = pl.kernel(
        _body,
        out_type=jax.ShapeDtypeStruct((B * T * D,), jnp.float32),
        mesh=mesh,
        scratch_types=[
            pltpu.VMEM((S + L,), jnp.int32),
            pltpu.VMEM((CH, D), jnp.float32),
            pltpu.VMEM(((TR + 1) * D,), jnp.float32),
        ],
    )(hidden.reshape(B * S, D), idx32.reshape(B * S))
    return out.reshape(B, T, D)


def kernel(hidden, ori_indexes):
    return _aggregate(hidden, ori_indexes.astype(jnp.int32))


# speedup vs baseline: 1.5929x; 1.0040x over previous
import jax
import jax.numpy as jnp
from jax import lax
from jax.experimental import pallas as pl
from jax.experimental.pallas import tpu as pltpu
from jax.experimental.pallas import tpu_sc as plsc

B, S, D = 16, 2048, 128
T = 1024
NC, NS = 2, 16
L = 16
TR = 256
NASSIGN = (B * T) // TR
CH = 128


def _body(hid, idx, out, idxv, datab, acc):
    c = lax.axis_index("c")
    w = lax.axis_index("s")
    tid = c * NS + w

    for a_rel in range(NASSIGN // (NC * NS)):
        a = tid * (NASSIGN // (NC * NS)) + a_rel
        b = a // (T // TR)
        t0 = (a % (T // TR)) * TR

        pltpu.sync_copy(idx.at[pl.ds(b * S, S)], idxv.at[pl.ds(0, S)])
        idxv[pl.ds(S, L)] = jnp.full((L,), jnp.int32(2**30), jnp.int32)

        def _search(target):
            def bs(_, lh):
                lo, hi = lh
                mid = (lo + hi) // 2
                v = idxv[pl.ds(mid, L)][0]
                go = v < target
                return (jnp.where(go, mid + 1, lo), jnp.where(go, hi, mid))
            lo, _ = lax.fori_loop(0, 12, bs, (jnp.int32(0), jnp.int32(S)))
            return lo

        lo_s = _search(t0)
        hi_s = _search(t0 + TR)
        k0 = lo_s // CH
        kmax = (hi_s + CH - 1) // CH

        def _za(r, _):
            for j in range(D // L):
                acc[pl.ds(r * D + j * L, L)] = jnp.zeros((L,), jnp.float32)
            return 0
        lax.fori_loop(0, TR, _za, 0)

        z = jnp.zeros((L,), jnp.float32)

        def _chunk(kk, carry):
            off = b * S + (k0 + kk) * CH
            pltpu.sync_copy(hid.at[pl.ds(off, CH)], datab)

            def _row(ib, cr):
                iv = idxv[pl.ds((k0 + kk) * CH + ib * L, L)]
                for k in range(L):
                    prev, ptarg, cf, a0, a1, a2, a3, a4, a5, a6, a7 = cr
                    t = iv[k]
                    new = t != prev
                    rel = t - t0
                    targ = jnp.where(jnp.logical_and(rel >= 0, rel < TR), rel, TR)

                    @pl.when(new)
                    def _flush(vals=(a0, a1, a2, a3, a4, a5, a6, a7),
                               pt=ptarg, pc=cf):
                        rcp = jnp.ones((L,), jnp.float32) / jnp.full((L,), pc, jnp.float32)
                        for j, aj in enumerate(vals):
                            acc[pl.ds(pt * D + j * L, L)] = aj * rcp

                    sf = jnp.where(new, 0.0, 1.0)
                    i = ib * L + k
                    avs = []
                    for j, aj in enumerate((a0, a1, a2, a3, a4, a5, a6, a7)):
                        avs.append(datab[i, pl.ds(j * L, L)] + aj * sf)
                    cr = (t, targ, 1.0 + cf * sf, *avs)
                return cr

            return lax.fori_loop(0, CH // L, _row, carry)

        fin = lax.fori_loop(0, kmax - k0, _chunk,
                            (jnp.int32(-1), jnp.int32(TR), jnp.float32(1.0),
                             z, z, z, z, z, z, z, z))
        frcp = jnp.ones((L,), jnp.float32) / jnp.full((L,), fin[2], jnp.float32)
        for j in range(D // L):
            acc[pl.ds(fin[1] * D + j * L, L)] = fin[3 + j] * frcp

        pltpu.sync_copy(acc.at[pl.ds(0, TR * D)], out.at[pl.ds((b * T + t0) * D, TR * D)])


@jax.jit
def _aggregate(hidden, idx32):
    mesh = plsc.VectorSubcoreMesh(
        core_axis_name="c", subcore_axis_name="s", num_cores=NC, num_subcores=NS
    )
    out = pl.kernel(
        _body,
        out_type=jax.ShapeDtypeStruct((B * T * D,), jnp.float32),
        mesh=mesh,
        scratch_types=[
            pltpu.VMEM((S + L,), jnp.int32),
            pltpu.VMEM((CH, D), jnp.float32),
            pltpu.VMEM(((TR + 1) * D,), jnp.float32),
        ],
    )(hidden.reshape(B * S, D), idx32.reshape(B * S))
    return out.reshape(B, T, D)


def kernel(hidden, ori_indexes):
    return _aggregate(hidden, ori_indexes.astype(jnp.int32))
